# baseline (speedup 1.0000x reference)
import functools

import jax
import jax.numpy as jnp
from jax import lax
from jax.experimental import pallas as pl
from jax.experimental.pallas import tpu as pltpu
from jax.experimental.pallas import tpu_sc as plsc

_Z = 64
_K = 1024
_N = 6272
_BN = 896
_NPAD = 6656
_BPW = 208
_NW = 32
_BETA = 0.25


def _conv(x, w, b, stride, pad):
    y = lax.conv_general_dilated(
        x, w, (stride, stride), [(pad, pad), (pad, pad)],
        dimension_numbers=('NCHW', 'OIHW', 'NCHW'))
    return y + b[None, :, None, None]


def _conv_t(x, w, b, stride, pad):
    k = w.shape[2]
    wt = jnp.flip(w, (2, 3)).transpose(1, 0, 2, 3)
    y = lax.conv_general_dilated(
        x, wt, (1, 1), [(k - 1 - pad, k - 1 - pad), (k - 1 - pad, k - 1 - pad)],
        lhs_dilation=(stride, stride), dimension_numbers=('NCHW', 'OIHW', 'NCHW'))
    return y + b[None, :, None, None]


def _res(x, w1, b1, w2, b2):
    out = jax.nn.relu(_conv(x, w1, b1, 1, 1))
    out = _conv(out, w2, b2, 1, 1)
    return jax.nn.relu(out + x)


def _argmin_body(e_ref, cb_ref, idx_ref):
    e = e_ref[...]
    cb = cb_ref[...]
    en2 = jnp.sum(e ** 2, axis=1, keepdims=True)
    c2 = jnp.sum(cb ** 2, axis=1)
    m = lax.dot_general(e, cb, (((1,), (1,)), ((), ())),
                        preferred_element_type=jnp.float32)
    d2 = en2 + c2[None, :] - 2.0 * m
    dist = jnp.sqrt(jnp.maximum(d2, 0.0))
    mn = jnp.min(dist, axis=1, keepdims=True)
    ids = lax.broadcasted_iota(jnp.int32, dist.shape, 1)
    idx = jnp.min(jnp.where(dist == mn, ids, _K), axis=1, keepdims=True)
    idx_ref[...] = idx


def _codebook_argmin(e_flat, codebook):
    out = pl.pallas_call(
        _argmin_body,
        grid=(_N // _BN,),
        in_specs=[
            pl.BlockSpec((_BN, _Z), lambda i: (i, 0)),
            pl.BlockSpec((_K, _Z), lambda i: (0, 0)),
        ],
        out_specs=pl.BlockSpec((_BN, 1), lambda i: (i, 0)),
        out_shape=jax.ShapeDtypeStruct((_N, 1), jnp.int32),
    )(e_flat, codebook)
    return out[:, 0]


_KH = _K + 8
_W = 128


def _sc_body(idxg_hbm, idxh_hbm, cb_hbm, ones_hbm, zeros_hbm, q_hbm, cnt_hbm,
             idxg_v, idxh_v, rows_v, ones_v, shared, sem):
    cid = lax.axis_index("c")
    sid = lax.axis_index("s")
    wid = sid * 2 + cid
    r0 = 2 * wid
    pltpu.sync_copy(idxg_hbm.at[pl.ds(r0, 2)], idxg_v)
    pltpu.sync_copy(idxh_hbm.at[pl.ds(r0, 2)], idxh_v)
    pltpu.sync_copy(ones_hbm, ones_v)
    cp0 = pltpu.async_copy(cb_hbm.at[idxg_v.at[0]], rows_v.at[0], sem)
    cp1 = pltpu.async_copy(cb_hbm.at[idxg_v.at[1]], rows_v.at[1], sem)
    cp0.wait()
    cp1.wait()
    pltpu.sync_copy(rows_v, q_hbm.at[pl.ds(r0, 2)])

    @pl.when(sid == 0)
    def _zero():
        pltpu.sync_copy(zeros_hbm, shared)
    plsc.subcore_barrier()
    pltpu.sync_copy(ones_v.at[0], shared.at[idxh_v.at[0]], add=True)
    pltpu.sync_copy(ones_v.at[1], shared.at[idxh_v.at[1]], add=True)
    plsc.subcore_barrier()

    @pl.when(sid == 0)
    def _writeback():
        pltpu.sync_copy(shared, cnt_hbm.at[cid])


def _sc_gather_hist(idx_pad, idx_hist, codebook):
    mesh = plsc.VectorSubcoreMesh(core_axis_name="c", subcore_axis_name="s")
    fn = functools.partial(
        pl.kernel,
        mesh=mesh,
        out_type=[
            jax.ShapeDtypeStruct((_NW * 2, 104, _W), jnp.float32),
            jax.ShapeDtypeStruct((2, _KH, _W), jnp.float32),
        ],
        scratch_types=[
            pltpu.VMEM((2, 104), jnp.int32),
            pltpu.VMEM((2, 104), jnp.int32),
            pltpu.VMEM((2, 104, _W), jnp.float32),
            pltpu.VMEM((2, 104, _W), jnp.float32),
            pltpu.VMEM_SHARED((_KH, _W), jnp.float32),
            pltpu.SemaphoreType.DMA,
        ],
    )(_sc_body)
    cb_pad = jnp.pad(codebook, ((0, 0), (0, _W - _Z)))
    ones = jnp.ones((2, 104, _W), jnp.float32)
    zeros = jnp.zeros((_KH, _W), jnp.float32)
    q_pad, cnt = fn(idx_pad.reshape(_NW * 2, 104),
                    idx_hist.reshape(_NW * 2, 104), cb_pad, ones, zeros)
    return q_pad.reshape(_NPAD, _W)[:_N, :_Z], cnt[:, :_K, 0]


def _vq_body(e_ref, q_ref, cnt_ref, zq_ref, vq_ref, perp_ref):
    e = e_ref[...]
    q = q_ref[...]
    en = jnp.sqrt(jnp.sum(e * e, axis=1, keepdims=True))
    qn = jnp.sqrt(jnp.sum(q * q, axis=1, keepdims=True))
    e_norm = e / jnp.maximum(en, 1e-12)
    c_norm = q / jnp.maximum(qn, 1e-12)
    v = e_norm - c_norm
    vn = jnp.sqrt(jnp.sum(v * v, axis=1, keepdims=True))
    msk = (vn > 1e-5).astype(jnp.float32)
    v = msk * (v / (vn + 1e-5)) + (1.0 - msk) * v
    zqt = e - 2.0 * v * jnp.sum(v * e, axis=1, keepdims=True)
    zq_ref[...] = e + (zqt - e)
    diff = e - q
    vq_ref[...] = jnp.full((1, 1), jnp.sum(diff * diff) / float(_N * _Z))
    counts = jnp.sum(cnt_ref[...], axis=0)
    avg = counts / float(_N)
    perp_ref[...] = jnp.full((1, 1), jnp.exp(-jnp.sum(avg * jnp.log(avg + 1e-10))))


def _vq_transform(e_flat, q_flat, cnt):
    return pl.pallas_call(
        _vq_body,
        out_shape=[
            jax.ShapeDtypeStruct((_N, _Z), jnp.float32),
            jax.ShapeDtypeStruct((1, 1), jnp.float32),
            jax.ShapeDtypeStruct((1, 1), jnp.float32),
        ],
    )(e_flat, q_flat, cnt)


def _loss_body(xr_ref, x_ref, vq_ref, rec_ref, tot_ref):
    dd = xr_ref[...] - x_ref[...]
    npix = float(xr_ref.shape[0] * xr_ref.shape[1])
    rec = jnp.full((1, 1), jnp.sum(dd * dd) / npix)
    vq = vq_ref[...]
    rec_ref[...] = rec
    tot_ref[...] = rec + vq + _BETA * vq


def _losses(x_recon, x, vq):
    xr2 = x_recon.reshape(2352, 128)
    x2 = x.reshape(2352, 128)
    return pl.pallas_call(
        _loss_body,
        out_shape=[
            jax.ShapeDtypeStruct((1, 1), jnp.float32),
            jax.ShapeDtypeStruct((1, 1), jnp.float32),
        ],
    )(xr2, x2, vq)


def kernel(x, codebook, ec1w, ec1b, ec2w, ec2b, er1w1, er1b1, er1w2, er1b2,
           er2w1, er2b1, er2w2, er2b2, dr1w1, dr1b1, dr1w2, dr1b2, dr2w1,
           dr2b1, dr2w2, dr2b2, dt1w, dt1b, dt2w, dt2b):
    z = jax.nn.relu(_conv(x, ec1w, ec1b, 2, 1))
    z = jax.nn.relu(_conv(z, ec2w, ec2b, 2, 1))
    z = _res(z, er1w1, er1b1, er1w2, er1b2)
    z_e = _res(z, er2w1, er2b1, er2w2, er2b2)
    b, c, h, w = z_e.shape
    e_flat = z_e.transpose(0, 2, 3, 1).reshape(-1, c)

    indices = _codebook_argmin(e_flat, codebook)
    idx_pad = jnp.pad(indices, (0, _NPAD - _N))
    idx_hist = jnp.pad(indices, (0, _NPAD - _N), constant_values=_K)
    q_flat, cnt = _sc_gather_hist(idx_pad, idx_hist, codebook)
    zq_st, vq, perp = _vq_transform(e_flat, q_flat, cnt)

    z_q_st = zq_st.reshape(b, h, w, c).transpose(0, 3, 1, 2)
    d = _res(z_q_st, dr1w1, dr1b1, dr1w2, dr1b2)
    d = _res(d, dr2w1, dr2b1, dr2w2, dr2b2)
    d = jax.nn.relu(_conv_t(d, dt1w, dt1b, 2, 1))
    x_recon = jnp.tanh(_conv_t(d, dt2w, dt2b, 2, 1))

    rec, tot = _losses(x_recon, x, vq)
    return (x_recon, tot[0, 0], rec[0, 0], vq[0, 0], vq[0, 0], perp[0, 0],
            indices.reshape(b, h, w))


# speedup vs baseline: 1.1455x; 1.1455x over previous
import functools

import jax
import jax.numpy as jnp
from jax import lax
from jax.experimental import pallas as pl
from jax.experimental.pallas import tpu as pltpu
from jax.experimental.pallas import tpu_sc as plsc

_Z = 64
_K = 1024
_N = 6272
_BN = 896
_NPAD = 6656
_NW = 32
_KH = _K + 8
_W = 128
_BETA = 0.25


def _conv(x, w, b, stride, pad):
    y = lax.conv_general_dilated(
        x, w, (stride, stride), [(pad, pad), (pad, pad)],
        dimension_numbers=('NCHW', 'OIHW', 'NCHW'))
    return y + b[None, :, None, None]


def _conv_t(x, w, b, stride, pad):
    k = w.shape[2]
    wt = jnp.flip(w, (2, 3)).transpose(1, 0, 2, 3)
    y = lax.conv_general_dilated(
        x, wt, (1, 1), [(k - 1 - pad, k - 1 - pad), (k - 1 - pad, k - 1 - pad)],
        lhs_dilation=(stride, stride), dimension_numbers=('NCHW', 'OIHW', 'NCHW'))
    return y + b[None, :, None, None]


def _res(x, w1, b1, w2, b2):
    out = jax.nn.relu(_conv(x, w1, b1, 1, 1))
    out = _conv(out, w2, b2, 1, 1)
    return jax.nn.relu(out + x)


def _vq_core_body(e_ref, cb_ref, zq_ref, idx_ref, vq_ref):
    e = e_ref[...]
    cb = cb_ref[...]
    en2 = jnp.sum(e ** 2, axis=1, keepdims=True)
    c2 = jnp.sum(cb ** 2, axis=1)
    m = lax.dot_general(e, cb, (((1,), (1,)), ((), ())),
                        preferred_element_type=jnp.float32)
    dist = jnp.sqrt(jnp.maximum(en2 + c2[None, :] - 2.0 * m, 0.0))
    mn = jnp.min(dist, axis=1, keepdims=True)
    ids = lax.broadcasted_iota(jnp.int32, dist.shape, 1)
    idx = jnp.min(jnp.where(dist == mn, ids, _K), axis=1, keepdims=True)
    idx_ref[...] = idx
    oh = (ids == idx).astype(jnp.float32)
    q = lax.dot_general(oh, cb, (((1,), (0,)), ((), ())),
                        preferred_element_type=jnp.float32)
    en = jnp.sqrt(en2)
    qn = jnp.sqrt(jnp.sum(q * q, axis=1, keepdims=True))
    e_nrm = e / jnp.maximum(en, 1e-12)
    c_nrm = q / jnp.maximum(qn, 1e-12)
    v = e_nrm - c_nrm
    vn = jnp.sqrt(jnp.sum(v * v, axis=1, keepdims=True))
    w = (vn > 1e-5).astype(jnp.float32)
    v = w * (v / (vn + 1e-5)) + (1.0 - w) * v
    zqt = e - 2.0 * v * jnp.sum(v * e, axis=1, keepdims=True)
    zq_ref[...] = e + (zqt - e)
    diff = e - q

    @pl.when(pl.program_id(0) == 0)
    def _init():
        vq_ref[...] = jnp.zeros_like(vq_ref)

    vq_ref[...] += jnp.full((1, 1), jnp.sum(diff * diff))


def _vq_core(e_flat, codebook):
    zq, idx, vq = pl.pallas_call(
        _vq_core_body,
        grid=(_N // _BN,),
        in_specs=[
            pl.BlockSpec((_BN, _Z), lambda i: (i, 0)),
            pl.BlockSpec((_K, _Z), lambda i: (0, 0)),
        ],
        out_specs=[
            pl.BlockSpec((_BN, _Z), lambda i: (i, 0)),
            pl.BlockSpec((_BN, 1), lambda i: (i, 0)),
            pl.BlockSpec((1, 1), lambda i: (0, 0)),
        ],
        out_shape=[
            jax.ShapeDtypeStruct((_N, _Z), jnp.float32),
            jax.ShapeDtypeStruct((_N, 1), jnp.int32),
            jax.ShapeDtypeStruct((1, 1), jnp.float32),
        ],
    )(e_flat, codebook)
    return zq, idx[:, 0], vq / float(_N * _Z)


def _sc_body(idxh_hbm, ones_hbm, zeros_hbm, cnt_hbm, idxh_v, ones_v, shared,
             sem):
    cid = lax.axis_index("c")
    sid = lax.axis_index("s")
    wid = sid * 2 + cid
    pltpu.sync_copy(idxh_hbm.at[pl.ds(2 * wid, 2)], idxh_v)
    pltpu.sync_copy(ones_hbm, ones_v)

    @pl.when(sid == 0)
    def _zero():
        pltpu.sync_copy(zeros_hbm, shared)
    plsc.subcore_barrier()
    pltpu.sync_copy(ones_v.at[0], shared.at[idxh_v.at[0]], add=True)
    pltpu.sync_copy(ones_v.at[1], shared.at[idxh_v.at[1]], add=True)
    plsc.subcore_barrier()

    @pl.when(sid == 0)
    def _writeback():
        pltpu.sync_copy(shared, cnt_hbm.at[cid])


def _sc_hist(idx_hist):
    mesh = plsc.VectorSubcoreMesh(core_axis_name="c", subcore_axis_name="s")
    fn = functools.partial(
        pl.kernel,
        mesh=mesh,
        out_type=jax.ShapeDtypeStruct((2, _KH, _W), jnp.float32),
        scratch_types=[
            pltpu.VMEM((2, 104), jnp.int32),
            pltpu.VMEM((2, 104, _W), jnp.float32),
            pltpu.VMEM_SHARED((_KH, _W), jnp.float32),
            pltpu.SemaphoreType.DMA,
        ],
    )(_sc_body)
    ones = jnp.ones((2, 104, _W), jnp.float32)
    zeros = jnp.zeros((_KH, _W), jnp.float32)
    cnt = fn(idx_hist.reshape(_NW * 2, 104), ones, zeros)
    return cnt[:, :_K, 0]


def _loss_body(xr_ref, x_ref, vq_ref, cnt_ref, rec_ref, tot_ref, perp_ref):
    dd = xr_ref[...] - x_ref[...]
    npix = float(xr_ref.shape[0] * xr_ref.shape[1])
    rec = jnp.full((1, 1), jnp.sum(dd * dd) / npix)
    vq = vq_ref[...]
    rec_ref[...] = rec
    tot_ref[...] = rec + vq + _BETA * vq
    counts = jnp.sum(cnt_ref[...], axis=0)
    avg = counts / float(_N)
    perp_ref[...] = jnp.full((1, 1), jnp.exp(-jnp.sum(avg * jnp.log(avg + 1e-10))))


def _losses(x_recon, x, vq, cnt):
    xr2 = x_recon.reshape(2352, 128)
    x2 = x.reshape(2352, 128)
    return pl.pallas_call(
        _loss_body,
        out_shape=[
            jax.ShapeDtypeStruct((1, 1), jnp.float32),
            jax.ShapeDtypeStruct((1, 1), jnp.float32),
            jax.ShapeDtypeStruct((1, 1), jnp.float32),
        ],
    )(xr2, x2, vq, cnt)


def kernel(x, codebook, ec1w, ec1b, ec2w, ec2b, er1w1, er1b1, er1w2, er1b2,
           er2w1, er2b1, er2w2, er2b2, dr1w1, dr1b1, dr1w2, dr1b2, dr2w1,
           dr2b1, dr2w2, dr2b2, dt1w, dt1b, dt2w, dt2b):
    z = jax.nn.relu(_conv(x, ec1w, ec1b, 2, 1))
    z = jax.nn.relu(_conv(z, ec2w, ec2b, 2, 1))
    z = _res(z, er1w1, er1b1, er1w2, er1b2)
    z_e = _res(z, er2w1, er2b1, er2w2, er2b2)
    b, c, h, w = z_e.shape
    e_flat = z_e.transpose(0, 2, 3, 1).reshape(-1, c)

    zq_st, indices, vq = _vq_core(e_flat, codebook)
    idx_hist = jnp.pad(indices, (0, _NPAD - _N), constant_values=_K)
    cnt = _sc_hist(idx_hist)

    z_q_st = zq_st.reshape(b, h, w, c).transpose(0, 3, 1, 2)
    d = _res(z_q_st, dr1w1, dr1b1, dr1w2, dr1b2)
    d = _res(d, dr2w1, dr2b1, dr2w2, dr2b2)
    d = jax.nn.relu(_conv_t(d, dt1w, dt1b, 2, 1))
    x_recon = jnp.tanh(_conv_t(d, dt2w, dt2b, 2, 1))

    rec, tot, perp = _losses(x_recon, x, vq, cnt)
    return (x_recon, tot[0, 0], rec[0, 0], vq[0, 0], vq[0, 0], perp[0, 0],
            indices.reshape(b, h, w))
